# baseline (speedup 1.0000x reference)
import functools

import jax
import jax.numpy as jnp
from jax import lax
from jax.experimental import pallas as pl
from jax.experimental.pallas import tpu as pltpu
from jax.experimental.pallas import tpu_sc as plsc

N = 10000
E = 320000
D = 128
R = 8

XW = 160
BF = jnp.bfloat16

NSC = 2
NT = 16
NTT = NSC * NT
K = 128
EPT = 10240
EPAD = NTT * EPT
NGRP = 4
CAP16 = 656
SHIFT = 14

ACC_R = 2 * N + 16
TRASH = 2 * N
ZPT = ACC_R // NT
CRT = 2 * N // NT
TRASH_DESC = TRASH << SHIFT

DN = lax.GatherDimensionNumbers(
    offset_dims=(), collapsed_slice_dims=(0,), start_index_map=(0,))


def _vgather(vals, idx):
    return lax.gather(vals, idx[:, None], DN, slice_sizes=(1,),
                      mode=lax.GatherScatterMode.PROMISE_IN_BOUNDS)


def _pack_body(dst_ref, rel_ref, src_ref, o_ref):
    dv = dst_ref[...]
    rv = rel_ref[...]
    sv = src_ref[...]
    ok = rv < R
    g = jnp.where(ok, rv >> 1, 0)
    row = jnp.where(ok, (rv & 1) * N + dv, TRASH)
    o_ref[...] = (g << 29) | (row << SHIFT) | sv


def _build_packed(dst_p, rel_p, src_p):
    return pl.pallas_call(
        _pack_body,
        out_shape=jax.ShapeDtypeStruct((NTT, EPT), jnp.int32),
    )(dst_p, rel_p, src_p)


def _bin_body(packed_hbm, binned_hbm, nbs_hbm, pk2, bv, nbv, sem):
    c = lax.axis_index("c")
    s = lax.axis_index("s")
    t = c * NT + s
    lanes = lax.iota(jnp.int32, 16)
    trash16 = jnp.full((16,), TRASH_DESC, jnp.int32)

    def tf_body(i, _):
        bv[i, pl.ds(0, 16)] = trash16
        return 0
    lax.fori_loop(0, NGRP * CAP16, tf_body, 0)

    pltpu.sync_copy(packed_hbm.at[t], pk2)

    def step(i, carry):
        ptrs, pcnts, pends = carry
        pv = pk2[pl.ds(i * 16, 16)]
        gs = pv >> 29
        nptr, npc, npd = [], [], []
        for g in range(NGRP):
            ptr, pcnt, pend = ptrs[g], pcnts[g], pends[g]
            mi = jnp.where(gs == g, 1, 0)
            cs = mi
            for sh in (1, 2, 4, 8):
                gsh = _vgather(cs, jnp.maximum(lanes - sh, 0))
                cs = cs + jnp.where(lanes >= sh, gsh, 0)
            cnt = cs[15]
            pos = jnp.zeros((16,), jnp.int32)
            for sh in (8, 4, 2, 1):
                cand = pos + sh
                cv = _vgather(cs, cand - 1)
                pos = jnp.where(cv < lanes + 1, cand, pos)
            comp = _vgather(pv, jnp.minimum(pos, 15))
            rot = _vgather(comp, (lanes - pcnt) & 15)
            merged = jnp.where(lanes < pcnt, pend, rot)
            bv[g * CAP16 + ptr, pl.ds(0, 16)] = merged
            total = pcnt + cnt
            full = (total >= 16).astype(jnp.int32)
            rot2 = _vgather(comp, (lanes + (16 - pcnt)) & 15)
            pend2 = jnp.where(total >= 16, rot2, merged)
            nptr.append(ptr + full)
            npc.append(total - 16 * full)
            npd.append(pend2)
        return tuple(nptr), tuple(npc), tuple(npd)

    init = (tuple(jnp.int32(0) for _ in range(NGRP)),
            tuple(jnp.int32(0) for _ in range(NGRP)),
            tuple(jnp.zeros((16,), jnp.int32) for _ in range(NGRP)))
    ptrs, pcnts, pends = lax.fori_loop(0, EPT // 16, step, init)

    nbvec = jnp.zeros((16,), jnp.int32)
    for g in range(NGRP):
        flush = jnp.where(lanes < pcnts[g], pends[g], TRASH_DESC)
        bv[g * CAP16 + ptrs[g], pl.ds(0, 16)] = flush
        ln = ptrs[g] * 16 + pcnts[g]
        nbb = jnp.maximum(((ln + 255) // 256) * 2, 2)
        nbvec = jnp.where(lanes == g, nbb, nbvec)
    nbv[pl.ds(0, 16)] = nbvec
    pltpu.sync_copy(bv, binned_hbm.at[t])
    pltpu.sync_copy(nbv, nbs_hbm.at[t])


_sc_bin = pl.kernel(
    _bin_body,
    out_type=(jax.ShapeDtypeStruct((NTT, NGRP * CAP16, 16), jnp.int32),
              jax.ShapeDtypeStruct((NTT, 16), jnp.int32)),
    mesh=plsc.VectorSubcoreMesh(
        core_axis_name="c", subcore_axis_name="s",
        num_cores=NSC, num_subcores=NT),
    scratch_types=[
        pltpu.VMEM((EPT,), jnp.int32),
        pltpu.VMEM((NGRP * CAP16, 16), jnp.int32),
        pltpu.VMEM((16,), jnp.int32),
        pltpu.SemaphoreType.DMA,
    ],
    compiler_params=pltpu.CompilerParams(use_tc_tiling_on_sc=False),
)


def _sc_agg_body(x_hbm, binned_hbm, nbs_hbm, zeros_hbm, agg_hbm,
                 db0, db1, ib0, ib1, sb0, sb1, rows0, rows1, nbv, acc,
                 gsem0, gsem1):
    c = lax.axis_index("c")
    s = lax.axis_index("s")
    lanes = lax.iota(jnp.int32, 16)
    dbs = (db0, db1)
    ibs = (ib0, ib1)
    sbs = (sb0, sb1)
    rows = (rows0, rows1)
    gsems = (gsem0, gsem1)

    def load_unpack(t2, g, b, par):
        pltpu.sync_copy(binned_hbm.at[t2, pl.ds(g * CAP16 + b * 8, 8)],
                        dbs[par])
        for j in range(8):
            pv = dbs[par][j, pl.ds(0, 16)]
            ibs[par][pl.ds(j * 16, 16)] = (pv >> SHIFT) & 0x7FFF
            sbs[par][pl.ds(j * 16, 16)] = pv & ((1 << SHIFT) - 1)

    def job(t2, g):
        pltpu.sync_copy(nbs_hbm.at[t2], nbv)
        nbrot = _vgather(nbv[pl.ds(0, 16)], (lanes + g) & 15)
        nb = nbrot[0]
        load_unpack(t2, g, 0, 0)
        pltpu.async_copy(x_hbm.at[sb0], rows0, gsem0)

        def bb_body(bb, _):
            for par in range(2):
                b = bb * 2 + par
                pltpu.make_async_copy(x_hbm.at[sbs[par]], rows[par],
                                      gsems[par]).wait()
                load_unpack(t2, g, b + 1, 1 - par)
                pltpu.async_copy(x_hbm.at[sbs[1 - par]], rows[1 - par],
                                 gsems[1 - par])
                pltpu.sync_copy(rows[par], acc.at[ibs[par]], add=True)
            return 0
        lax.fori_loop(0, nb >> 1, bb_body, 0)
        pltpu.make_async_copy(x_hbm.at[sb0], rows0, gsem0).wait()

    for gp in range(2):
        g = c * 2 + gp

        pltpu.sync_copy(zeros_hbm, acc.at[pl.ds(s * ZPT, ZPT)])
        plsc.subcore_barrier()

        job(s * 2, g)
        job(s * 2 + 1, g)
        plsc.subcore_barrier()

        rr = s // 8
        n0 = (s % 8) * CRT
        pltpu.sync_copy(acc.at[pl.ds(s * CRT, CRT)],
                        agg_hbm.at[2 * g + rr, pl.ds(n0, CRT)])
        plsc.subcore_barrier()


_sc_agg = pl.kernel(
    _sc_agg_body,
    out_type=jax.ShapeDtypeStruct((R, N, XW), BF),
    mesh=plsc.VectorSubcoreMesh(
        core_axis_name="c", subcore_axis_name="s",
        num_cores=NSC, num_subcores=NT),
    scratch_types=[
        pltpu.VMEM((8, 16), jnp.int32),
        pltpu.VMEM((8, 16), jnp.int32),
        pltpu.VMEM((K,), jnp.int32),
        pltpu.VMEM((K,), jnp.int32),
        pltpu.VMEM((K,), jnp.int32),
        pltpu.VMEM((K,), jnp.int32),
        pltpu.VMEM((K, XW), BF),
        pltpu.VMEM((K, XW), BF),
        pltpu.VMEM((16,), jnp.int32),
        pltpu.VMEM_SHARED((ACC_R, XW), BF),
        pltpu.SemaphoreType.DMA,
        pltpu.SemaphoreType.DMA,
    ],
    compiler_params=pltpu.CompilerParams(use_tc_tiling_on_sc=False),
)


def _tc_layer_body(apply_gelu, x_ref, agg_ref, w_ref, root_ref, bias_ref,
                   o_ref):
    xfull = x_ref[:, :D].astype(jnp.float32)
    acc = jnp.dot(xfull, root_ref[...], preferred_element_type=jnp.float32)
    for r in range(R):
        a = agg_ref[r][:, :D].astype(jnp.float32)
        cnt = agg_ref[r][:, D:D + 1].astype(jnp.float32)
        scale = 1.0 / jnp.maximum(cnt, 1.0)
        acc += jnp.dot(a * scale, w_ref[r],
                       preferred_element_type=jnp.float32)
    acc = acc + bias_ref[...]
    if apply_gelu:
        acc = acc * 0.5 * (1.0 + lax.erf(acc * (2.0 ** -0.5)))
        y = jnp.concatenate(
            [acc.astype(BF), jnp.ones((acc.shape[0], 1), BF),
             jnp.zeros((acc.shape[0], XW - D - 1), BF)], axis=1)
        o_ref[...] = y
    else:
        o_ref[...] = acc


def _tc_layer(x, agg, weight, root, bias, apply_gelu):
    BN = 400
    ow, odt = (XW, BF) if apply_gelu else (D, jnp.float32)
    return pl.pallas_call(
        functools.partial(_tc_layer_body, apply_gelu),
        grid=(N // BN,),
        in_specs=[
            pl.BlockSpec((BN, XW), lambda i: (i, 0)),
            pl.BlockSpec((R, BN, XW), lambda i: (0, i, 0)),
            pl.BlockSpec((R, D, D), lambda i: (0, 0, 0)),
            pl.BlockSpec((D, D), lambda i: (0, 0)),
            pl.BlockSpec((1, D), lambda i: (0, 0)),
        ],
        out_specs=pl.BlockSpec((BN, ow), lambda i: (i, 0)),
        out_shape=jax.ShapeDtypeStruct((N, ow), odt),
    )(x, agg, weight, root, bias)


def kernel(embs, edge_index, rel_type, batch_size, weight1, root1, bias1,
           weight2, root2, bias2):
    src = edge_index[0]
    dst = edge_index[1]
    pad = EPAD - E
    src_p = jnp.concatenate(
        [src, jnp.zeros((pad,), jnp.int32)]).reshape(NTT, EPT)
    dst_p = jnp.concatenate(
        [dst, jnp.zeros((pad,), jnp.int32)]).reshape(NTT, EPT)
    rel_p = jnp.concatenate(
        [rel_type, jnp.full((pad,), R, jnp.int32)]).reshape(NTT, EPT)
    packed = _build_packed(dst_p, rel_p, src_p)
    binned, nbs = _sc_bin(packed)
    zeros_acc = jnp.zeros((ZPT, XW), BF)

    x0 = jnp.concatenate(
        [embs.astype(BF), jnp.ones((N, 1), BF),
         jnp.zeros((N, XW - D - 1), BF)], axis=1)

    agg1 = _sc_agg(x0, binned, nbs, zeros_acc)
    x1 = _tc_layer(x0, agg1, weight1, root1, bias1.reshape(1, D), True)
    agg2 = _sc_agg(x1, binned, nbs, zeros_acc)
    out = _tc_layer(x1, agg2, weight2, root2, bias2.reshape(1, D), False)
    return out


# speedup vs baseline: 1.5893x; 1.5893x over previous
import functools

import jax
import jax.numpy as jnp
from jax import lax
from jax.experimental import pallas as pl
from jax.experimental.pallas import tpu as pltpu
from jax.experimental.pallas import tpu_sc as plsc

N = 10000
E = 320000
D = 128
R = 8

CW = 32
NFC = D // CW
NCC = NFC + 1
BF = jnp.bfloat16

NSC = 2
NT = 16
K = 128
NB = 80
EPT = NB * K
EPAD = NSC * NT * EPT
SHIFT = 14

ACC_R = R * N + 16
TRASH = R * N
ZPT = ACC_R // NT
CRT = R * N // NT


def _pack_body(dst_ref, rel_ref, src_ref, o_ref):
    dv = dst_ref[...]
    rv = rel_ref[...]
    sv = src_ref[...]
    row = jnp.where(rv < R, rv * N + dv, TRASH)
    o_ref[...] = (row << SHIFT) | sv


def _build_packed(dst_p, rel_p, src_p):
    return pl.pallas_call(
        _pack_body,
        out_shape=jax.ShapeDtypeStruct((NSC, NT, NB, K), jnp.int32),
    )(dst_p, rel_p, src_p)


def _sc_agg_body(with_counts, xa, xb, xc, xd, packed_hbm, zeros_hbm,
                 ones_hbm, agg_hbm, packed2, rows0, rows1, onesb, sb0, sb1,
                 ib0, ib1, acc, gsem0, gsem1):
    c = lax.axis_index("c")
    s = lax.axis_index("s")
    rows = (rows0, rows1)
    sbufs = (sb0, sb1)
    ibufs = (ib0, ib1)
    gsems = (gsem0, gsem1)

    pltpu.sync_copy(ones_hbm, onesb)
    pltpu.sync_copy(packed_hbm.at[c, s], packed2)

    def unpack(b, par, need_src):
        for j in range(K // 16):
            pv = packed2[b, pl.ds(j * 16, 16)]
            ibufs[par][pl.ds(j * 16, 16)] = pv >> SHIFT
            if need_src:
                sbufs[par][pl.ds(j * 16, 16)] = pv & ((1 << SHIFT) - 1)

    def copyout(cc):
        rr = s // 2
        n0 = (s % 2) * CRT
        pltpu.sync_copy(acc.at[pl.ds(s * CRT, CRT)],
                        agg_hbm.at[cc, c, rr, pl.ds(n0, CRT)])

    for cc in range(NFC):
        xin = (xa, xb, xc, xd)[cc]

        pltpu.sync_copy(zeros_hbm, acc.at[pl.ds(s * ZPT, ZPT)])
        plsc.subcore_barrier()

        unpack(0, 0, True)
        pltpu.async_copy(xin.at[sb0], rows0, gsem0)

        def bb_body(bb, _):
            for par in range(2):
                b = bb * 2 + par
                rbuf, obuf = rows[par], rows[1 - par]
                pltpu.make_async_copy(xin.at[sbufs[par]], rbuf,
                                      gsems[par]).wait()
                unpack(b + 1, 1 - par, True)
                pltpu.async_copy(xin.at[sbufs[1 - par]], obuf,
                                 gsems[1 - par])
                pltpu.sync_copy(rbuf, acc.at[ibufs[par]], add=True)
            return 0
        lax.fori_loop(0, (NB - 2) // 2, bb_body, 0)
        pltpu.make_async_copy(xin.at[sb0], rows0, gsem0).wait()
        unpack(NB - 1, 1, True)
        pltpu.async_copy(xin.at[sb1], rows1, gsem1)
        pltpu.sync_copy(rows0, acc.at[ib0], add=True)
        pltpu.make_async_copy(xin.at[sb1], rows1, gsem1).wait()
        pltpu.sync_copy(rows1, acc.at[ib1], add=True)
        plsc.subcore_barrier()

        copyout(cc)
        plsc.subcore_barrier()

    if with_counts:
        pltpu.sync_copy(zeros_hbm, acc.at[pl.ds(s * ZPT, ZPT)])
        plsc.subcore_barrier()

        def cb_body(b, _):
            unpack(b, 0, False)
            pltpu.sync_copy(onesb, acc.at[ib0], add=True)
            return 0
        lax.fori_loop(0, NB, cb_body, 0)
        plsc.subcore_barrier()
        copyout(NFC)
        plsc.subcore_barrier()


def _make_sc_agg(with_counts):
    return pl.kernel(
        functools.partial(_sc_agg_body, with_counts),
        out_type=jax.ShapeDtypeStruct(
            (NCC if with_counts else NFC, NSC, R, N, CW), BF),
        mesh=plsc.VectorSubcoreMesh(
            core_axis_name="c", subcore_axis_name="s",
            num_cores=NSC, num_subcores=NT),
        scratch_types=[
            pltpu.VMEM((NB, K), jnp.int32),
            pltpu.VMEM((K, CW), BF),
            pltpu.VMEM((K, CW), BF),
            pltpu.VMEM((K, CW), BF),
            pltpu.VMEM((K,), jnp.int32),
            pltpu.VMEM((K,), jnp.int32),
            pltpu.VMEM((K,), jnp.int32),
            pltpu.VMEM((K,), jnp.int32),
            pltpu.VMEM_SHARED((ACC_R, CW), BF),
            pltpu.SemaphoreType.DMA,
            pltpu.SemaphoreType.DMA,
        ],
        compiler_params=pltpu.CompilerParams(use_tc_tiling_on_sc=False),
    )


_sc_agg_c = _make_sc_agg(True)
_sc_agg_nc = _make_sc_agg(False)


def _tc_layer_body(apply_gelu, xa_ref, xb_ref, xc_ref, xd_ref, agg_ref,
                   cnt_ref, w_ref, root_ref, bias_ref, *out_refs):
    xfull = jnp.concatenate(
        [xa_ref[...], xb_ref[...], xc_ref[...], xd_ref[...]],
        axis=1).astype(jnp.float32)
    acc = jnp.dot(xfull, root_ref[...], preferred_element_type=jnp.float32)
    for r in range(R):
        a = jnp.concatenate(
            [agg_ref[q, 0, r].astype(jnp.float32)
             + agg_ref[q, 1, r].astype(jnp.float32) for q in range(NFC)],
            axis=1)
        if apply_gelu:
            cnt = (agg_ref[NFC, 0, r][:, 0:1].astype(jnp.float32)
                   + agg_ref[NFC, 1, r][:, 0:1].astype(jnp.float32))
        else:
            cnt = (cnt_ref[0, r][:, 0:1].astype(jnp.float32)
                   + cnt_ref[1, r][:, 0:1].astype(jnp.float32))
        scale = 1.0 / jnp.maximum(cnt, 1.0)
        acc += jnp.dot(a * scale, w_ref[r],
                       preferred_element_type=jnp.float32)
    acc = acc + bias_ref[...]
    if apply_gelu:
        acc = acc * 0.5 * (1.0 + lax.erf(acc * (2.0 ** -0.5)))
        for q in range(NFC):
            out_refs[q][...] = acc[:, q * CW:(q + 1) * CW].astype(BF)
    else:
        out_refs[0][...] = acc


def _tc_layer(xs, agg, cnts, weight, root, bias, apply_gelu):
    BN = 400
    nag = NCC if apply_gelu else NFC
    if apply_gelu:
        out_shape = [jax.ShapeDtypeStruct((N, CW), BF)] * NFC
        out_specs = [pl.BlockSpec((BN, CW), lambda i: (i, 0))] * NFC
    else:
        out_shape = jax.ShapeDtypeStruct((N, D), jnp.float32)
        out_specs = pl.BlockSpec((BN, D), lambda i: (i, 0))
    return pl.pallas_call(
        functools.partial(_tc_layer_body, apply_gelu),
        grid=(N // BN,),
        in_specs=[
            pl.BlockSpec((BN, CW), lambda i: (i, 0)),
            pl.BlockSpec((BN, CW), lambda i: (i, 0)),
            pl.BlockSpec((BN, CW), lambda i: (i, 0)),
            pl.BlockSpec((BN, CW), lambda i: (i, 0)),
            pl.BlockSpec((nag, NSC, R, BN, CW), lambda i: (0, 0, 0, i, 0)),
            pl.BlockSpec((NSC, R, BN, CW), lambda i: (0, 0, i, 0)),
            pl.BlockSpec((R, D, D), lambda i: (0, 0, 0)),
            pl.BlockSpec((D, D), lambda i: (0, 0)),
            pl.BlockSpec((1, D), lambda i: (0, 0)),
        ],
        out_specs=out_specs,
        out_shape=out_shape,
    )(*xs, agg, cnts, weight, root, bias)


def kernel(embs, edge_index, rel_type, batch_size, weight1, root1, bias1,
           weight2, root2, bias2):
    src = edge_index[0]
    dst = edge_index[1]
    pad = EPAD - E
    src_p = jnp.concatenate(
        [src, jnp.zeros((pad,), jnp.int32)]).reshape(NSC, NT, NB, K)
    dst_p = jnp.concatenate(
        [dst, jnp.zeros((pad,), jnp.int32)]).reshape(NSC, NT, NB, K)
    rel_p = jnp.concatenate(
        [rel_type, jnp.full((pad,), R, jnp.int32)]).reshape(NSC, NT, NB, K)
    packed = _build_packed(dst_p, rel_p, src_p)
    zeros_acc = jnp.zeros((ZPT, CW), BF)
    ones_col = jnp.zeros((K, CW), BF).at[:, 0].set(1)

    xs = tuple(embs[:, q * CW:(q + 1) * CW].astype(BF) for q in range(NFC))

    agg1 = _sc_agg_c(*xs, packed, zeros_acc, ones_col)
    cnts = agg1[NFC]
    xs1 = _tc_layer(xs, agg1, cnts, weight1, root1,
                    bias1.reshape(1, D), True)
    agg2 = _sc_agg_nc(*xs1, packed, zeros_acc, ones_col)
    out = _tc_layer(xs1, agg2, cnts, weight2, root2,
                    bias2.reshape(1, D), False)
    return out
